# baseline (speedup 1.0000x reference)
import jax
import jax.numpy as jnp
from jax.experimental import pallas as pl
from jax.experimental.pallas import tpu as pltpu

B = 4096
IN_DIMS = 1024
HID = 1024
OUT = 1024
BLK = 512
NB = B // BLK


def _body(x_ref, ws_ref, bs_ref, g0_ref, b0_ref, we_ref, be_ref, out_ref,
          h_ref, acc_ref, wsb_ref, web_ref, s_ref, c_ref):
    p = pl.program_id(0)
    i = pl.program_id(1)

    @pl.when(jnp.logical_and(p == 0, i == 0))
    def _init0():
        wsb_ref[...] = ws_ref[...].astype(jnp.bfloat16)

    @pl.when(p == 0)
    def _phase0():
        xb = x_ref[...].astype(jnp.bfloat16)
        h = jnp.dot(xb, wsb_ref[...], preferred_element_type=jnp.float32)
        h = jnp.maximum(h + bs_ref[...], 0.0)
        h_ref[pl.ds(i * BLK, BLK), :] = h.astype(jnp.bfloat16)
        psum = jnp.sum(h, axis=0, keepdims=True)
        psq = jnp.sum(h * h, axis=0, keepdims=True)
        blk_acc = jnp.concatenate([psum, psq], axis=0)

        @pl.when(i == 0)
        def _():
            acc_ref[...] = blk_acc

        @pl.when(i > 0)
        def _():
            acc_ref[...] += blk_acc

    @pl.when(p == 1)
    def _phase1():
        @pl.when(i == 0)
        def _init1():
            web_ref[...] = we_ref[...].astype(jnp.bfloat16)
            m = acc_ref[0:1, :] / B
            v = acc_ref[1:2, :] / B - m * m
            s = g0_ref[...] * jax.lax.rsqrt(v + 1e-5)
            s_ref[...] = s.astype(jnp.bfloat16)
            t = (b0_ref[...] - m * s).astype(jnp.bfloat16)
            c = jnp.dot(t, web_ref[...], preferred_element_type=jnp.float32)
            c_ref[...] = c + be_ref[...]

        hn = h_ref[pl.ds(i * BLK, BLK), :] * s_ref[...]
        o = jnp.dot(hn, web_ref[...], preferred_element_type=jnp.float32)
        out_ref[...] = jnp.maximum(o + c_ref[...], 0.0)


def kernel(x, W_start, b_start, bn0_g, bn0_b, W_exp, b_exp, bn_g, bn_b,
           W_end, b_end, W_dqn, b_dqn):
    del W_exp, b_exp, bn_g, bn_b, W_dqn, b_dqn
    bs = b_start.reshape(1, HID)
    g0 = bn0_g.reshape(1, HID)
    b0 = bn0_b.reshape(1, HID)
    be = b_end.reshape(1, OUT)
    return pl.pallas_call(
        _body,
        grid=(2, NB),
        in_specs=[
            pl.BlockSpec((BLK, IN_DIMS), lambda p, i: (i * (1 - p), 0)),
            pl.BlockSpec((IN_DIMS, HID), lambda p, i: (0, 0)),
            pl.BlockSpec((1, HID), lambda p, i: (0, 0)),
            pl.BlockSpec((1, HID), lambda p, i: (0, 0)),
            pl.BlockSpec((1, HID), lambda p, i: (0, 0)),
            pl.BlockSpec((HID, OUT), lambda p, i: (0, 0)),
            pl.BlockSpec((1, OUT), lambda p, i: (0, 0)),
        ],
        out_specs=pl.BlockSpec((BLK, OUT), lambda p, i: (i * p, 0)),
        out_shape=jax.ShapeDtypeStruct((B, OUT), jnp.float32),
        scratch_shapes=[
            pltpu.VMEM((B, HID), jnp.bfloat16),
            pltpu.VMEM((2, HID), jnp.float32),
            pltpu.VMEM((IN_DIMS, HID), jnp.bfloat16),
            pltpu.VMEM((HID, OUT), jnp.bfloat16),
            pltpu.VMEM((1, HID), jnp.bfloat16),
            pltpu.VMEM((1, OUT), jnp.float32),
        ],
        compiler_params=pltpu.CompilerParams(
            dimension_semantics=("arbitrary", "arbitrary")),
    )(x, W_start, bs, g0, b0, W_end, be)


# speedup vs baseline: 1.0503x; 1.0503x over previous
import jax
import jax.numpy as jnp
from jax.experimental import pallas as pl
from jax.experimental.pallas import tpu as pltpu

B = 4096
IN_DIMS = 1024
HID = 1024
OUT = 1024
BLK = 1024
NB = B // BLK


def _body(x_ref, ws_ref, bs_ref, g0_ref, b0_ref, we_ref, be_ref, out_ref,
          h_ref, acc_ref, wsb_ref, web_ref, s_ref, c_ref):
    p = pl.program_id(0)
    i = pl.program_id(1)

    @pl.when(jnp.logical_and(p == 0, i == 0))
    def _init0():
        wsb_ref[...] = ws_ref[...].astype(jnp.bfloat16)

    @pl.when(p == 0)
    def _phase0():
        xb = x_ref[...].astype(jnp.bfloat16)
        h = jnp.dot(xb, wsb_ref[...], preferred_element_type=jnp.float32)
        h = jnp.maximum(h + bs_ref[...], 0.0)
        h_ref[pl.ds(i * BLK, BLK), :] = h.astype(jnp.bfloat16)
        psum = jnp.sum(h, axis=0, keepdims=True)
        psq = jnp.sum(h * h, axis=0, keepdims=True)
        blk_acc = jnp.concatenate([psum, psq], axis=0)

        @pl.when(i == 0)
        def _():
            acc_ref[...] = blk_acc

        @pl.when(i > 0)
        def _():
            acc_ref[...] += blk_acc

    @pl.when(p == 1)
    def _phase1():
        @pl.when(i == 0)
        def _init1():
            web_ref[...] = we_ref[...].astype(jnp.bfloat16)
            m = acc_ref[0:1, :] / B
            v = acc_ref[1:2, :] / B - m * m
            s = g0_ref[...] * jax.lax.rsqrt(v + 1e-5)
            s_ref[...] = s.astype(jnp.bfloat16)
            t = (b0_ref[...] - m * s).astype(jnp.bfloat16)
            c = jnp.dot(t, web_ref[...], preferred_element_type=jnp.float32)
            c_ref[...] = c + be_ref[...]

        hn = h_ref[pl.ds(i * BLK, BLK), :] * s_ref[...]
        o = jnp.dot(hn, web_ref[...], preferred_element_type=jnp.float32)
        out_ref[...] = jnp.maximum(o + c_ref[...], 0.0)


def kernel(x, W_start, b_start, bn0_g, bn0_b, W_exp, b_exp, bn_g, bn_b,
           W_end, b_end, W_dqn, b_dqn):
    del W_exp, b_exp, bn_g, bn_b, W_dqn, b_dqn
    bs = b_start.reshape(1, HID)
    g0 = bn0_g.reshape(1, HID)
    b0 = bn0_b.reshape(1, HID)
    be = b_end.reshape(1, OUT)
    return pl.pallas_call(
        _body,
        grid=(2, NB),
        in_specs=[
            pl.BlockSpec((BLK, IN_DIMS), lambda p, i: (i * (1 - p), 0)),
            pl.BlockSpec((IN_DIMS, HID), lambda p, i: (0, 0)),
            pl.BlockSpec((1, HID), lambda p, i: (0, 0)),
            pl.BlockSpec((1, HID), lambda p, i: (0, 0)),
            pl.BlockSpec((1, HID), lambda p, i: (0, 0)),
            pl.BlockSpec((HID, OUT), lambda p, i: (0, 0)),
            pl.BlockSpec((1, OUT), lambda p, i: (0, 0)),
        ],
        out_specs=pl.BlockSpec((BLK, OUT), lambda p, i: (i * p, 0)),
        out_shape=jax.ShapeDtypeStruct((B, OUT), jnp.float32),
        scratch_shapes=[
            pltpu.VMEM((B, HID), jnp.bfloat16),
            pltpu.VMEM((2, HID), jnp.float32),
            pltpu.VMEM((IN_DIMS, HID), jnp.bfloat16),
            pltpu.VMEM((HID, OUT), jnp.bfloat16),
            pltpu.VMEM((1, HID), jnp.bfloat16),
            pltpu.VMEM((1, OUT), jnp.float32),
        ],
        compiler_params=pltpu.CompilerParams(
            dimension_semantics=("arbitrary", "arbitrary")),
    )(x, W_start, bs, g0, b0, W_end, be)


# speedup vs baseline: 1.0763x; 1.0247x over previous
import jax
import jax.numpy as jnp
from jax.experimental import pallas as pl
from jax.experimental.pallas import tpu as pltpu

B = 4096
IN_DIMS = 1024
HID = 1024
OUT = 1024
CH = 512
NCH = B // CH


def _body(x_hbm, ws_hbm, bs_ref, g0_ref, b0_ref, we_hbm, be_ref, out_hbm,
          xbuf, obuf, h_ref, wsf_ref, wef_ref, wsb_ref, web_ref,
          in_sems, w_sems, out_sems):
    in_cps = [
        pltpu.make_async_copy(x_hbm.at[pl.ds(c * CH, CH), :],
                              xbuf.at[c % 2], in_sems.at[c % 2])
        for c in range(NCH)
    ]
    ws_cp = pltpu.make_async_copy(ws_hbm, wsf_ref, w_sems.at[0])
    we_cp = pltpu.make_async_copy(we_hbm, wef_ref, w_sems.at[1])
    ws_cp.start()
    in_cps[0].start()
    in_cps[1].start()
    we_cp.start()
    ws_cp.wait()
    wsb_ref[...] = wsf_ref[...].astype(jnp.bfloat16)

    ps = jnp.zeros((1, HID), jnp.float32)
    pq = jnp.zeros((1, HID), jnp.float32)
    for c in range(NCH):
        in_cps[c].wait()
        xb = xbuf[c % 2].astype(jnp.bfloat16)
        h = jnp.dot(xb, wsb_ref[...], preferred_element_type=jnp.float32)
        h = jnp.maximum(h + bs_ref[...], 0.0)
        h_ref[pl.ds(c * CH, CH), :] = h.astype(jnp.bfloat16)
        ps = ps + jnp.sum(h, axis=0, keepdims=True)
        pq = pq + jnp.sum(h * h, axis=0, keepdims=True)
        if c + 2 < NCH:
            in_cps[c + 2].start()

    we_cp.wait()
    web_ref[...] = wef_ref[...].astype(jnp.bfloat16)
    m = ps / B
    v = pq / B - m * m
    s = g0_ref[...] * jax.lax.rsqrt(v + 1e-5)
    sb = s.astype(jnp.bfloat16)
    t = (b0_ref[...] - m * s).astype(jnp.bfloat16)
    crow = jnp.dot(t, web_ref[...], preferred_element_type=jnp.float32)
    crow = crow + be_ref[...]

    out_cps = [
        pltpu.make_async_copy(obuf.at[c % 2],
                              out_hbm.at[pl.ds(c * CH, CH), :],
                              out_sems.at[c % 2])
        for c in range(NCH)
    ]
    for c in range(NCH):
        hn = h_ref[pl.ds(c * CH, CH), :] * sb
        o = jnp.dot(hn, web_ref[...], preferred_element_type=jnp.float32)
        if c >= 2:
            out_cps[c - 2].wait()
        obuf[c % 2] = jnp.maximum(o + crow, 0.0)
        out_cps[c].start()
    out_cps[NCH - 2].wait()
    out_cps[NCH - 1].wait()


def kernel(x, W_start, b_start, bn0_g, bn0_b, W_exp, b_exp, bn_g, bn_b,
           W_end, b_end, W_dqn, b_dqn):
    del W_exp, b_exp, bn_g, bn_b, W_dqn, b_dqn
    bs = b_start.reshape(1, HID)
    g0 = bn0_g.reshape(1, HID)
    b0 = bn0_b.reshape(1, HID)
    be = b_end.reshape(1, OUT)
    any_spec = pl.BlockSpec(memory_space=pltpu.MemorySpace.HBM)
    vmem_spec = pl.BlockSpec(memory_space=pltpu.MemorySpace.VMEM)
    return pl.pallas_call(
        _body,
        in_specs=[
            any_spec,
            any_spec,
            vmem_spec,
            vmem_spec,
            vmem_spec,
            any_spec,
            vmem_spec,
        ],
        out_specs=any_spec,
        out_shape=jax.ShapeDtypeStruct((B, OUT), jnp.float32),
        scratch_shapes=[
            pltpu.VMEM((2, CH, IN_DIMS), jnp.float32),
            pltpu.VMEM((2, CH, OUT), jnp.float32),
            pltpu.VMEM((B, HID), jnp.bfloat16),
            pltpu.VMEM((IN_DIMS, HID), jnp.float32),
            pltpu.VMEM((HID, OUT), jnp.float32),
            pltpu.VMEM((IN_DIMS, HID), jnp.bfloat16),
            pltpu.VMEM((HID, OUT), jnp.bfloat16),
            pltpu.SemaphoreType.DMA((2,)),
            pltpu.SemaphoreType.DMA((2,)),
            pltpu.SemaphoreType.DMA((2,)),
        ],
    )(x, W_start, bs, g0, b0, W_end, be)
